# baseline (speedup 1.0000x reference)
import jax
import jax.numpy as jnp
from jax.experimental import pallas as pl
from jax.experimental.pallas import tpu as pltpu


def _appnp_block(alpha_ref, adj_ref, x_ref, x0_ref, out_ref):
    a = alpha_ref[0]
    prop = jnp.dot(
        adj_ref[...].astype(jnp.bfloat16),
        x_ref[...].astype(jnp.bfloat16),
        preferred_element_type=jnp.float32,
    )
    out_ref[...] = a * prop + (1.0 - a) * x0_ref[...]


def kernel(x, adj, x_0, alpha):
    N, d = x.shape
    BM = 512
    return pl.pallas_call(
        _appnp_block,
        grid=(pl.cdiv(N, BM),),
        in_specs=[
            pl.BlockSpec(memory_space=pltpu.SMEM),
            pl.BlockSpec((BM, N), lambda i: (i, 0)),
            pl.BlockSpec((N, d), lambda i: (0, 0)),
            pl.BlockSpec((BM, d), lambda i: (i, 0)),
        ],
        out_specs=pl.BlockSpec((BM, d), lambda i: (i, 0)),
        out_shape=jax.ShapeDtypeStruct((N, d), jnp.float32),
    )(alpha, adj, x, x_0)


# speedup vs baseline: 1.0136x; 1.0136x over previous
import jax
import jax.numpy as jnp
from jax.experimental import pallas as pl
from jax.experimental.pallas import tpu as pltpu


def _appnp_block(alpha_ref, adj_ref, x_ref, x0_ref, out_ref):
    a = alpha_ref[0]
    prop = jnp.dot(
        adj_ref[...].astype(jnp.bfloat16),
        x_ref[...].astype(jnp.bfloat16),
        preferred_element_type=jnp.float32,
    )
    out_ref[...] = a * prop + (1.0 - a) * x0_ref[...]


def kernel(x, adj, x_0, alpha):
    N, d = x.shape
    BM = 256
    return pl.pallas_call(
        _appnp_block,
        grid=(pl.cdiv(N, BM),),
        in_specs=[
            pl.BlockSpec(memory_space=pltpu.SMEM),
            pl.BlockSpec((BM, N), lambda i: (i, 0)),
            pl.BlockSpec((N, d), lambda i: (0, 0)),
            pl.BlockSpec((BM, d), lambda i: (i, 0)),
        ],
        out_specs=pl.BlockSpec((BM, d), lambda i: (i, 0)),
        out_shape=jax.ShapeDtypeStruct((N, d), jnp.float32),
    )(alpha, adj, x, x_0)
